# baseline (speedup 1.0000x reference)
import jax
import jax.numpy as jnp
from jax.experimental import pallas as pl
from jax.experimental.pallas import tpu as pltpu

_B = 4000
_K = 16


def _insert_body(state_hbm, maskv_ref, masks_ref, ns_hbm,
                 out_hbm, outm_ref, carry_ref, sems, gsem):
    i = pl.program_id(0)
    ng = pl.num_programs(0)
    nb = ns_hbm.shape[0]
    b = _B

    @pl.when(i == 0)
    def _():
        carry_ref[0] = 0
        carry_ref[2] = 0

    c0 = carry_ref[0]
    m2 = maskv_ref[0]
    e2 = (~m2).astype(jnp.int32)
    zeros = jnp.sum(e2)

    cond_copy = jnp.logical_or(c0 >= nb, zeros == 0)
    cond_fast = jnp.logical_and(zeros == b, c0 + b <= nb)
    cond_gen = jnp.logical_not(jnp.logical_or(cond_copy, cond_fast))

    def ring_dma(src_ref):
        nring = carry_ref[2]
        slot = jax.lax.rem(nring, _K)
        for k in range(_K):
            @pl.when(slot == k)
            def _():
                cp = pltpu.make_async_copy(
                    src_ref,
                    out_hbm.at[pl.ds(i * b, b), :],
                    sems.at[k])

                @pl.when(nring >= _K)
                def _():
                    cp.wait()

                cp.start()
        carry_ref[2] = nring + 1

    @pl.when(cond_copy)
    def _():
        outm_ref[...] = maskv_ref[...]
        ring_dma(state_hbm.at[pl.ds(i * b, b), :])

    @pl.when(cond_fast)
    def _():
        outm_ref[...] = jnp.ones_like(outm_ref)
        ring_dma(ns_hbm.at[pl.ds(c0, b), :])

    @pl.when(cond_gen)
    def _():
        lane = jax.lax.broadcasted_iota(jnp.int32, (1, b), 1)
        x = e2
        off = 1
        while off < b:
            x = x + jnp.where(lane >= off, jnp.roll(x, off, axis=1), 0)
            off *= 2
        excl = x - e2
        cnt = c0 + excl
        ins = jnp.logical_and(e2 > 0, cnt < nb)
        outm_ref[...] = jnp.logical_or(m2, ins).reshape(outm_ref.shape)

        cpb = pltpu.make_async_copy(
            state_hbm.at[pl.ds(i * b, b), :],
            out_hbm.at[pl.ds(i * b, b), :],
            gsem)
        cpb.start()
        cpb.wait()

        fo = jnp.min(jnp.where(m2, lane, b))
        run = jnp.minimum(fo, nb - c0)
        for k in range(11, -1, -1):
            sz = 1 << k
            done = (run >> (k + 1)) << (k + 1)

            @pl.when(((run >> k) & 1) == 1)
            def _():
                cp = pltpu.make_async_copy(
                    ns_hbm.at[pl.ds(c0 + done, sz), :],
                    out_hbm.at[pl.ds(i * b + done, sz), :],
                    gsem)
                cp.start()
                cp.wait()

        carry_ref[1] = c0 + run

        def row_body(r, _):
            em = masks_ref[0, 0, r] == 0
            c = carry_ref[1]

            @pl.when(jnp.logical_and(em, c < nb))
            def _():
                cp = pltpu.make_async_copy(
                    ns_hbm.at[pl.ds(c, 1), :],
                    out_hbm.at[pl.ds(i * b + r, 1), :],
                    gsem)
                cp.start()
                cp.wait()

            @pl.when(em)
            def _():
                carry_ref[1] = c + 1

            return 0

        jax.lax.fori_loop(run, b, row_body, 0)

    carry_ref[0] = c0 + zeros

    @pl.when(i == ng - 1)
    def _():
        ntot = carry_ref[2]
        for k in range(_K):
            @pl.when(k < ntot)
            def _():
                pltpu.make_async_copy(
                    ns_hbm.at[pl.ds(0, b), :],
                    out_hbm.at[pl.ds(0, b), :],
                    sems.at[k]).wait()


def kernel(state, mask, new_states):
    m, d = state.shape
    nb = new_states.shape[0]
    g = m // _B
    mask3 = mask.reshape(g, 1, _B)
    mask3_i32 = mask3.astype(jnp.int32)

    out_state, out_mask3 = pl.pallas_call(
        _insert_body,
        grid=(g,),
        in_specs=[
            pl.BlockSpec(memory_space=pl.ANY),
            pl.BlockSpec((1, 1, _B), lambda i: (i, 0, 0)),
            pl.BlockSpec((1, 1, _B), lambda i: (i, 0, 0),
                         memory_space=pltpu.SMEM),
            pl.BlockSpec(memory_space=pl.ANY),
        ],
        out_specs=[
            pl.BlockSpec(memory_space=pl.ANY),
            pl.BlockSpec((1, 1, _B), lambda i: (i, 0, 0)),
        ],
        out_shape=[
            jax.ShapeDtypeStruct((m, d), state.dtype),
            jax.ShapeDtypeStruct((g, 1, _B), jnp.bool_),
        ],
        scratch_shapes=[
            pltpu.SMEM((4,), jnp.int32),
            pltpu.SemaphoreType.DMA((_K,)),
            pltpu.SemaphoreType.DMA,
        ],
    )(state, mask3, mask3_i32, new_states)
    return out_state, out_mask3.reshape(m)


# speedup vs baseline: 16.7911x; 16.7911x over previous
import jax
import jax.numpy as jnp
from jax.experimental import pallas as pl
from jax.experimental.pallas import tpu as pltpu

_B = 20000
_CS = 2000


def _insert_body(state_ref, maskv_ref, maskw_hbm, ns_ref,
                 out_ref, outm_ref, carry_ref, mchunk_ref, dsem):
    i = pl.program_id(0)
    nb = ns_ref.shape[0]
    b = _B

    @pl.when(i == 0)
    def _():
        carry_ref[0] = 0

    c0 = carry_ref[0]
    m2 = maskv_ref[0]
    e2 = (~m2).astype(jnp.int32)
    zeros = jnp.sum(e2)

    cond_copy = jnp.logical_or(c0 >= nb, zeros == 0)
    cond_fast = jnp.logical_and(zeros == b, c0 + b <= nb)
    cond_gen = jnp.logical_not(jnp.logical_or(cond_copy, cond_fast))

    @pl.when(cond_copy)
    def _():
        out_ref[...] = state_ref[...]
        outm_ref[...] = maskv_ref[...]

    @pl.when(cond_fast)
    def _():
        out_ref[...] = ns_ref[pl.ds(c0, b), :]
        outm_ref[...] = jnp.ones_like(outm_ref)

    @pl.when(cond_gen)
    def _():
        out_ref[...] = state_ref[...]
        lane = jax.lax.broadcasted_iota(jnp.int32, (1, b), 1)
        x = e2
        off = 1
        while off < b:
            x = x + jnp.where(lane >= off, jnp.roll(x, off, axis=1), 0)
            off *= 2
        excl = x - e2
        cnt = c0 + excl
        ins = jnp.logical_and(e2 > 0, cnt < nb)
        outm_ref[...] = jnp.logical_or(m2, ins).reshape(outm_ref.shape)

        fo = jnp.min(jnp.where(m2, lane, b))
        run = jnp.minimum(fo, nb - c0)
        for k in range(14, -1, -1):
            sz = 1 << k
            if sz > b or sz > nb:
                continue
            done = (run >> (k + 1)) << (k + 1)

            @pl.when(((run >> k) & 1) == 1)
            def _():
                out_ref[pl.ds(done, sz), :] = ns_ref[pl.ds(c0 + done, sz), :]

        carry_ref[1] = c0 + run
        first_chunk = run // _CS

        def chunk_body(ci, _):
            @pl.when(jnp.logical_and(ci >= first_chunk, carry_ref[1] < nb))
            def _():
                fetch = mchunk_ref.shape[0]
                start = i * b + ci * _CS
                clamp = ((maskw_hbm.shape[0] - fetch) // 128) * 128
                astart = pl.multiple_of(
                    jnp.minimum((start // 128) * 128, clamp), 128)
                delta = start - astart
                cp = pltpu.make_async_copy(
                    maskw_hbm.at[pl.ds(astart, fetch)],
                    mchunk_ref,
                    dsem)
                cp.start()
                cp.wait()

                def row_body(r, _):
                    ra = ci * _CS + r
                    em = jnp.logical_and(mchunk_ref[delta + r] == 0,
                                         ra >= run)
                    c = carry_ref[1]

                    @pl.when(jnp.logical_and(em, c < nb))
                    def _():
                        out_ref[pl.ds(ra, 1), :] = ns_ref[pl.ds(c, 1), :]

                    @pl.when(em)
                    def _():
                        carry_ref[1] = c + 1

                    return 0

                jax.lax.fori_loop(0, _CS, row_body, 0)
            return 0

        jax.lax.fori_loop(0, b // _CS, chunk_body, 0)

    carry_ref[0] = c0 + zeros


def kernel(state, mask, new_states):
    m, d = state.shape
    nb = new_states.shape[0]
    g = m // _B
    mask3 = mask.reshape(g, 1, _B)
    mask_i32 = jnp.pad(mask.astype(jnp.int32), (0, (-m) % 128),
                       constant_values=1)

    out_state, out_mask3 = pl.pallas_call(
        _insert_body,
        grid=(g,),
        in_specs=[
            pl.BlockSpec((_B, d), lambda i: (i, 0)),
            pl.BlockSpec((1, 1, _B), lambda i: (i, 0, 0)),
            pl.BlockSpec(memory_space=pl.ANY),
            pl.BlockSpec((nb, d), lambda i: (0, 0)),
        ],
        out_specs=[
            pl.BlockSpec((_B, d), lambda i: (i, 0)),
            pl.BlockSpec((1, 1, _B), lambda i: (i, 0, 0)),
        ],
        out_shape=[
            jax.ShapeDtypeStruct((m, d), state.dtype),
            jax.ShapeDtypeStruct((g, 1, _B), jnp.bool_),
        ],
        scratch_shapes=[
            pltpu.SMEM((4,), jnp.int32),
            pltpu.SMEM((-(-(_CS + 128) // 128) * 128,), jnp.int32),
            pltpu.SemaphoreType.DMA,
        ],
    )(state, mask3, mask_i32, new_states)
    return out_state, out_mask3.reshape(m)
